# Initial kernel scaffold; baseline (speedup 1.0000x reference)
import jax
import jax.numpy as jnp
from jax.experimental import pallas as pl


def kernel(pos, edge_index, W_lin1, W_src1, W_dst1, W_pos1, b_pos1, W_lin2, W_src2, W_dst2, W_pos2, b_pos2):
    raise NotImplementedError("write your pallas kernel here")


# speedup vs baseline: 4.9663x; 4.9663x over previous
import functools

import jax
import jax.numpy as jnp
from jax import lax
from jax.experimental import pallas as pl
from jax.experimental.pallas import tpu as pltpu
from jax.experimental.pallas import tpu_sc as plsc

_F32 = jnp.float32

_BN = 512
_C = 256
_KSUB = _C // 128
_NTILE = 16
_NCH = 4


def _tc_tables(z, wt):
    np_, kp = z.shape

    def body(z_ref, w_ref, s_ref, t_ref):
        m = jnp.dot(z_ref[...], w_ref[...], preferred_element_type=_F32)
        for k in range(_NCH):
            s_ref[k] = m[:, 32 * k:32 * k + 32]
            t_ref[k] = m[:, 128 + 32 * k:160 + 32 * k]

    return pl.pallas_call(
        body,
        grid=(np_ // _BN,),
        in_specs=[
            pl.BlockSpec((_BN, kp), lambda i: (i, 0)),
            pl.BlockSpec((kp, 256), lambda i: (0, 0)),
        ],
        out_specs=[
            pl.BlockSpec((_NCH, _BN, 32), lambda i: (0, i, 0)),
            pl.BlockSpec((_NCH, _BN, 32), lambda i: (0, i, 0)),
        ],
        out_shape=[jax.ShapeDtypeStruct((_NCH, np_, 32), _F32)] * 2,
    )(z, wt)


def _edge_call(n_acc, np_, ep, s_flat, t_flat, src2, dst2):
    rt = n_acc // _NTILE
    rb = 56
    nb = rt // rb
    tr = (ep // 128) // _NTILE
    n_chunks = tr // _KSUB

    mesh = plsc.VectorSubcoreMesh(core_axis_name="c", subcore_axis_name="s")

    @functools.partial(
        pl.kernel,
        out_type=jax.ShapeDtypeStruct((_NCH * n_acc, 16), _F32),
        mesh=mesh,
        compiler_params=pltpu.CompilerParams(use_tc_tiling_on_sc=False),
        scratch_types=[
            pltpu.VMEM_SHARED((n_acc, 32), _F32),
            pltpu.VMEM((_KSUB, 128), jnp.int32),
            pltpu.VMEM((_KSUB, 128), jnp.int32),
            pltpu.VMEM((_KSUB, 128), jnp.int32),
            pltpu.VMEM((_KSUB, 128), jnp.int32),
            pltpu.VMEM((_C, 32), _F32),
            pltpu.VMEM((_C, 32), _F32),
            pltpu.VMEM((rb, 32), _F32),
            pltpu.VMEM((rb, 16), _F32),
            pltpu.SemaphoreType.DMA,
            pltpu.SemaphoreType.DMA,
        ],
    )
    def ek(s_hbm, t_hbm, src_hbm, dst_hbm, out_hbm,
           acc, idx_s, idx_d, idx_sg, idx_dg, srows, trows,
           accv, obuf, sem_g, sem_s):
        c = lax.axis_index("c")
        s = lax.axis_index("s")
        zero16 = jnp.zeros((16,), _F32)

        for j in range(2):
            k = 2 * c + j
            koff = k * np_

            def zbody(r, carry):
                accv[r, pl.ds(0, 16)] = zero16
                accv[r, pl.ds(16, 16)] = zero16
                return carry
            lax.fori_loop(0, rb, zbody, None)
            for b in range(nb):
                pltpu.sync_copy(accv, acc.at[pl.ds(s * rt + b * rb, rb)])
            plsc.subcore_barrier()

            def chunk(i, carry):
                row = s * tr + i * _KSUB
                pltpu.sync_copy(src_hbm.at[pl.ds(row, _KSUB)], idx_s)
                pltpu.sync_copy(dst_hbm.at[pl.ds(row, _KSUB)], idx_d)
                for ki in range(_KSUB):
                    for jj in range(8):
                        sl = pl.ds(jj * 16, 16)
                        idx_sg[ki, sl] = idx_s[ki, sl] + koff
                        idx_dg[ki, sl] = idx_d[ki, sl] + koff
                cps = []
                for ki in range(_KSUB):
                    cps.append(pltpu.async_copy(
                        s_hbm.at[idx_sg.at[ki]],
                        srows.at[pl.ds(ki * 128, 128)], sem_g))
                    cps.append(pltpu.async_copy(
                        t_hbm.at[idx_dg.at[ki]],
                        trows.at[pl.ds(ki * 128, 128)], sem_g))
                for cp in cps:
                    cp.wait()

                def ebody(e, ecarry):
                    lo = pl.ds(0, 16)
                    hi = pl.ds(16, 16)
                    ex = jnp.exp(trows[e, lo] - srows[e, lo])
                    num = ex * (srows[e, hi] + trows[e, hi])
                    srows[e, lo] = ex
                    srows[e, hi] = num
                    return ecarry
                lax.fori_loop(0, _C, ebody, None)

                scps = []
                for ki in range(_KSUB):
                    scps.append(pltpu.async_copy(
                        srows.at[pl.ds(ki * 128, 128)],
                        acc.at[idx_d.at[ki]], sem_s, add=True))
                for cp in scps:
                    cp.wait()
                return carry
            lax.fori_loop(0, n_chunks, chunk, None)
            plsc.subcore_barrier()

            for b in range(nb):
                base = s * rt + b * rb
                pltpu.sync_copy(acc.at[pl.ds(base, rb)], accv)

                def dbody(r, carry):
                    den = accv[r, pl.ds(0, 16)]
                    num = accv[r, pl.ds(16, 16)]
                    obuf[r, pl.ds(0, 16)] = num / (den + 1e-16)
                    return carry
                lax.fori_loop(0, rb, dbody, None)
                pltpu.sync_copy(obuf, out_hbm.at[pl.ds(k * n_acc + base, rb)])
            plsc.subcore_barrier()

    return ek(s_flat, t_flat, src2, dst2)


def _build_wt(W_lin, W_src, W_dst, W_pos, b_pos, kp):
    d = W_lin.shape[0]
    bm = jnp.concatenate([W_src, W_pos, jnp.zeros((d, 1), _F32)], axis=1)
    vm = jnp.concatenate([W_lin, -W_pos, jnp.zeros((d, 1), _F32)], axis=1)
    am = jnp.concatenate([W_dst, W_pos, b_pos[:, None]], axis=1)
    qm = jnp.concatenate([jnp.zeros_like(W_src), W_pos, b_pos[:, None]],
                         axis=1)
    dz = bm.shape[1]
    s_rows = jnp.stack([bm.reshape(_NCH, 16, dz), vm.reshape(_NCH, 16, dz)],
                       axis=1).reshape(8 * 16, dz)
    t_rows = jnp.stack([am.reshape(_NCH, 16, dz), qm.reshape(_NCH, 16, dz)],
                       axis=1).reshape(8 * 16, dz)
    wcat = jnp.concatenate([s_rows, t_rows], axis=0)
    return jnp.pad(wcat, ((0, 0), (0, kp - dz))).T


def kernel(pos, edge_index, W_lin1, W_src1, W_dst1, W_pos1, b_pos1,
           W_lin2, W_src2, W_dst2, W_pos2, b_pos2):
    n = pos.shape[0]
    e = edge_index.shape[1]
    np_ = -(-n // _BN) * _BN
    if np_ == n:
        np_ += _BN
    ep = -(-e // (_NTILE * _C)) * (_NTILE * _C)

    src = edge_index[0]
    dst = edge_index[1]
    pad_e = ep - e
    if pad_e:
        src = jnp.concatenate([src, jnp.full((pad_e,), n, jnp.int32)])
        dst = jnp.concatenate([dst, jnp.zeros((pad_e,), jnp.int32)])
    src2 = src.reshape(ep // 128, 128)
    dst2 = dst.reshape(ep // 128, 128)

    ones = jnp.ones((n, 1), _F32)

    def layer(x, W_lin, W_src, W_dst, W_pos, b_pos):
        dz = x.shape[1] + 3 + 1
        kp = -(-dz // 8) * 8
        z = jnp.concatenate([x, pos, ones], axis=1)
        z = jnp.pad(z, ((0, np_ - n), (0, kp - dz)))
        wt = _build_wt(W_lin, W_src, W_dst, W_pos, b_pos, kp)
        s4, t4 = _tc_tables(z, wt)
        t4 = t4.at[:, n, 0:16].set(-1e30)
        out4 = _edge_call(np_, np_, ep,
                          s4.reshape(_NCH * np_, 32),
                          t4.reshape(_NCH * np_, 32), src2, dst2)
        return (out4.reshape(_NCH, np_, 16)[:, :n]
                .transpose(1, 0, 2).reshape(n, 64))

    x1 = layer(pos, W_lin1, W_src1, W_dst1, W_pos1, b_pos1)
    return layer(x1, W_lin2, W_src2, W_dst2, W_pos2, b_pos2)


# speedup vs baseline: 10.0637x; 2.0264x over previous
import functools

import jax
import jax.numpy as jnp
from jax import lax
from jax.experimental import pallas as pl
from jax.experimental.pallas import tpu as pltpu
from jax.experimental.pallas import tpu_sc as plsc

_F32 = jnp.float32

_BN = 512
_NTILE = 16
_NCH = 4
_EPAD = _NTILE * 128 * 8


def _tc_tables(z, wt):
    np_, kp = z.shape

    def body(z_ref, w_ref, s_ref, t_ref):
        m = jnp.dot(z_ref[...], w_ref[...], preferred_element_type=_F32)
        for k in range(_NCH):
            s_ref[k] = m[:, 32 * k:32 * k + 32]
            t_ref[k] = m[:, 128 + 32 * k:160 + 32 * k]

    return pl.pallas_call(
        body,
        grid=(np_ // _BN,),
        in_specs=[
            pl.BlockSpec((_BN, kp), lambda i: (i, 0)),
            pl.BlockSpec((kp, 256), lambda i: (0, 0)),
        ],
        out_specs=[
            pl.BlockSpec((_NCH, _BN, 32), lambda i: (0, i, 0)),
            pl.BlockSpec((_NCH, _BN, 32), lambda i: (0, i, 0)),
        ],
        out_shape=[jax.ShapeDtypeStruct((_NCH, np_, 32), _F32)] * 2,
    )(z, wt)


def _edge_call(n_acc, np_, ep, s_flat, t_flat, src2, dst2):
    rt = n_acc // _NTILE
    rb = 56
    nb = rt // rb
    tr = (ep // 128) // _NTILE

    mesh = plsc.VectorSubcoreMesh(core_axis_name="c", subcore_axis_name="s")

    @functools.partial(
        pl.kernel,
        out_type=jax.ShapeDtypeStruct((_NCH * n_acc, 16), _F32),
        mesh=mesh,
        compiler_params=pltpu.CompilerParams(use_tc_tiling_on_sc=False),
        scratch_types=[
            pltpu.VMEM_SHARED((n_acc, 32), _F32),
            pltpu.VMEM((8, 128), jnp.int32),
            pltpu.VMEM((8, 128), jnp.int32),
            pltpu.VMEM((8, 128), jnp.int32),
            pltpu.VMEM((128, 32), _F32),
            pltpu.VMEM((128, 32), _F32),
            pltpu.VMEM((128, 32), _F32),
            pltpu.VMEM((128, 32), _F32),
            pltpu.VMEM((rb, 32), _F32),
            pltpu.VMEM((rb, 16), _F32),
            pltpu.SemaphoreType.DMA,
            pltpu.SemaphoreType.DMA,
        ],
    )
    def ek(s_hbm, t_hbm, src_hbm, dst_hbm, out_hbm,
           acc, idx_s8, idx_d8, idx_dg, srows_a, trows_a, srows_b, trows_b,
           accv, obuf, sem_g, sem_s):
        c = lax.axis_index("c")
        s = lax.axis_index("s")
        zero16 = jnp.zeros((16,), _F32)

        for j in range(2):
            k = 2 * c + j
            koff = k * np_

            def zbody(r, carry):
                accv[r, pl.ds(0, 16)] = zero16
                accv[r, pl.ds(16, 16)] = zero16
                return carry
            lax.fori_loop(0, rb, zbody, None)
            for b in range(nb):
                pltpu.sync_copy(accv, acc.at[pl.ds(s * rt + b * rb, rb)])
            plsc.subcore_barrier()

            bufs = [(srows_a, trows_a), (srows_b, trows_b)]

            def group(g, carry):
                row = s * tr + g * 8
                pltpu.sync_copy(src_hbm.at[pl.ds(row, 8)], idx_s8)
                pltpu.sync_copy(dst_hbm.at[pl.ds(row, 8)], idx_d8)
                for ki in range(8):
                    for jj in range(8):
                        sl = pl.ds(jj * 16, 16)
                        idx_s8[ki, sl] = idx_s8[ki, sl] + koff
                        idx_dg[ki, sl] = idx_d8[ki, sl] + koff

                gcp = [None, None]
                scp = [None, None]

                def fire(t):
                    sb, tb = bufs[t % 2]
                    gcp[t % 2] = (
                        pltpu.async_copy(s_hbm.at[idx_s8.at[t]], sb, sem_g),
                        pltpu.async_copy(t_hbm.at[idx_dg.at[t]], tb, sem_g))

                fire(0)
                for t in range(8):
                    st = t % 2
                    if t + 1 < 8:
                        if scp[1 - st] is not None:
                            scp[1 - st].wait()
                        fire(t + 1)
                    for cpx in gcp[st]:
                        cpx.wait()
                    sb, tb = bufs[st]

                    def ebody(e4, ecarry):
                        lo = pl.ds(0, 16)
                        hi = pl.ds(16, 16)
                        for u in range(4):
                            e = e4 * 4 + u
                            ex = jnp.exp(tb[e, lo] - sb[e, lo])
                            num = ex * (sb[e, hi] + tb[e, hi])
                            sb[e, lo] = ex
                            sb[e, hi] = num
                        return ecarry
                    lax.fori_loop(0, 32, ebody, None)
                    scp[st] = pltpu.async_copy(
                        sb, acc.at[idx_d8.at[t]], sem_s, add=True)
                for x in scp:
                    if x is not None:
                        x.wait()
                return carry
            lax.fori_loop(0, tr // 8, group, None)
            plsc.subcore_barrier()

            for b in range(nb):
                base = s * rt + b * rb
                pltpu.sync_copy(acc.at[pl.ds(base, rb)], accv)

                def dbody(r, carry):
                    den = accv[r, pl.ds(0, 16)]
                    num = accv[r, pl.ds(16, 16)]
                    obuf[r, pl.ds(0, 16)] = num / (den + 1e-16)
                    return carry
                lax.fori_loop(0, rb, dbody, None)
                pltpu.sync_copy(obuf, out_hbm.at[pl.ds(k * n_acc + base, rb)])
            plsc.subcore_barrier()

    return ek(s_flat, t_flat, src2, dst2)


def _build_wt(W_lin, W_src, W_dst, W_pos, b_pos, kp):
    d = W_lin.shape[0]
    bm = jnp.concatenate([W_src, W_pos, jnp.zeros((d, 1), _F32)], axis=1)
    vm = jnp.concatenate([W_lin, -W_pos, jnp.zeros((d, 1), _F32)], axis=1)
    am = jnp.concatenate([W_dst, W_pos, b_pos[:, None]], axis=1)
    qm = jnp.concatenate([jnp.zeros_like(W_src), W_pos, b_pos[:, None]],
                         axis=1)
    dz = bm.shape[1]
    s_rows = jnp.stack([bm.reshape(_NCH, 16, dz), vm.reshape(_NCH, 16, dz)],
                       axis=1).reshape(8 * 16, dz)
    t_rows = jnp.stack([am.reshape(_NCH, 16, dz), qm.reshape(_NCH, 16, dz)],
                       axis=1).reshape(8 * 16, dz)
    wcat = jnp.concatenate([s_rows, t_rows], axis=0)
    return jnp.pad(wcat, ((0, 0), (0, kp - dz))).T


def kernel(pos, edge_index, W_lin1, W_src1, W_dst1, W_pos1, b_pos1,
           W_lin2, W_src2, W_dst2, W_pos2, b_pos2):
    n = pos.shape[0]
    e = edge_index.shape[1]
    np_ = -(-n // _BN) * _BN
    if np_ == n:
        np_ += _BN
    ep = -(-e // _EPAD) * _EPAD

    src = edge_index[0]
    dst = edge_index[1]
    pad_e = ep - e
    if pad_e:
        src = jnp.concatenate([src, jnp.full((pad_e,), n, jnp.int32)])
        dst = jnp.concatenate([dst, jnp.zeros((pad_e,), jnp.int32)])
    src2 = src.reshape(ep // 128, 128)
    dst2 = dst.reshape(ep // 128, 128)

    ones = jnp.ones((n, 1), _F32)

    def layer(x, W_lin, W_src, W_dst, W_pos, b_pos):
        dz = x.shape[1] + 3 + 1
        kp = -(-dz // 8) * 8
        z = jnp.concatenate([x, pos, ones], axis=1)
        z = jnp.pad(z, ((0, np_ - n), (0, kp - dz)))
        wt = _build_wt(W_lin, W_src, W_dst, W_pos, b_pos, kp)
        s4, t4 = _tc_tables(z, wt)
        t4 = t4.at[:, n, 0:16].set(-1e30)
        out4 = _edge_call(np_, np_, ep,
                          s4.reshape(_NCH * np_, 32),
                          t4.reshape(_NCH * np_, 32), src2, dst2)
        return (out4.reshape(_NCH, np_, 16)[:, :n]
                .transpose(1, 0, 2).reshape(n, 64))

    x1 = layer(pos, W_lin1, W_src1, W_dst1, W_pos1, b_pos1)
    return layer(x1, W_lin2, W_src2, W_dst2, W_pos2, b_pos2)


# speedup vs baseline: 11.5419x; 1.1469x over previous
import functools

import jax
import jax.numpy as jnp
from jax import lax
from jax.experimental import pallas as pl
from jax.experimental.pallas import tpu as pltpu
from jax.experimental.pallas import tpu_sc as plsc

_F32 = jnp.float32

_BN = 512
_NTILE = 16
_NCH = 4
_EPAD = _NTILE * 128 * 8


def _tc_tables(z, wt):
    np_, kp = z.shape

    def body(z_ref, w_ref, s_ref, t_ref):
        m = jnp.dot(z_ref[...], w_ref[...], preferred_element_type=_F32)
        for k in range(_NCH):
            s_ref[k] = m[:, 32 * k:32 * k + 32]
            t_ref[k] = m[:, 128 + 32 * k:160 + 32 * k]

    return pl.pallas_call(
        body,
        grid=(np_ // _BN,),
        in_specs=[
            pl.BlockSpec((_BN, kp), lambda i: (i, 0)),
            pl.BlockSpec((kp, 256), lambda i: (0, 0)),
        ],
        out_specs=[
            pl.BlockSpec((_NCH, _BN, 32), lambda i: (0, i, 0)),
            pl.BlockSpec((_NCH, _BN, 32), lambda i: (0, i, 0)),
        ],
        out_shape=[jax.ShapeDtypeStruct((_NCH, np_, 32), _F32)] * 2,
    )(z, wt)


def _edge_call(n_acc, np_, ep, s_flat, t_flat, idx_all):
    rt = n_acc // _NTILE
    rb = 56
    nb = rt // rb
    nrows = ep // 128
    tr = nrows // _NTILE
    ng = tr // 8

    mesh = plsc.VectorSubcoreMesh(core_axis_name="c", subcore_axis_name="s")

    def make_ek(j):
        return functools.partial(
            pl.kernel,
            out_type=jax.ShapeDtypeStruct((2 * n_acc, 16), _F32),
            mesh=mesh,
            compiler_params=pltpu.CompilerParams(use_tc_tiling_on_sc=False),
            scratch_types=[
            pltpu.VMEM_SHARED((n_acc, 32), _F32),
            pltpu.VMEM((8, 3, 128), jnp.int32),
            pltpu.VMEM((8, 3, 128), jnp.int32),
            pltpu.VMEM((128, 32), _F32),
            pltpu.VMEM((128, 32), _F32),
            pltpu.VMEM((128, 32), _F32),
            pltpu.VMEM((128, 32), _F32),
            pltpu.VMEM((rb, 32), _F32),
            pltpu.VMEM((rb, 16), _F32),
            pltpu.SemaphoreType.DMA,
            pltpu.SemaphoreType.DMA,
            pltpu.SemaphoreType.DMA,
            pltpu.SemaphoreType.DMA,
            pltpu.SemaphoreType.DMA,
        ],
    )
    def make_body(j):
        def ek(s_hbm, t_hbm, idx_hbm, out_hbm,
               acc, ibuf0, ibuf1, srows_a, trows_a, srows_b, trows_b,
               accv, obuf, sem_i, gsem0, gsem1, ssem0, ssem1):
            c = lax.axis_index("c")
            s = lax.axis_index("s")
            zero16 = jnp.zeros((16,), _F32)
            bufs = [(srows_a, trows_a), (srows_b, trows_b)]
            ibufs = [ibuf0, ibuf1]
            gsems = [gsem0, gsem1]
            ssems = [ssem0, ssem1]
            k = 2 * c + j

            def zbody(r, carry):
                accv[r, pl.ds(0, 16)] = zero16
                accv[r, pl.ds(16, 16)] = zero16
                return carry
            lax.fori_loop(0, rb, zbody, None)
            for b in range(nb):
                pltpu.sync_copy(accv, acc.at[pl.ds(s * rt + b * rb, rb)])
            plsc.subcore_barrier()

            rowb = k * nrows + s * tr

            def g_idx(g):
                return idx_hbm.at[pl.ds(rowb + g * 8, 8)]

            def fire_gather(ib, t, sid):
                sb, tb = bufs[sid]
                pltpu.async_copy(s_hbm.at[ib.at[t, 0]], sb, gsems[sid])
                pltpu.async_copy(t_hbm.at[ib.at[t, 1]], tb, gsems[sid])

            def drain_gather(sid):
                sb, tb = bufs[sid]
                pltpu.make_async_copy(
                    s_hbm.at[pl.ds(0, 128)], sb, gsems[sid]).wait()
                pltpu.make_async_copy(
                    s_hbm.at[pl.ds(0, 128)], tb, gsems[sid]).wait()

            def fire_scatter(ib, t, sid):
                sb = bufs[sid][0]
                pltpu.async_copy(sb, acc.at[ib.at[t, 2]], ssems[sid],
                                 add=True)

            def drain_scatter(sid):
                sb = bufs[sid][0]
                pltpu.make_async_copy(
                    sb, acc.at[pl.ds(0, 128)], ssems[sid]).wait()

            def do_group(g, ib, ib_next, first, last):
                if first:
                    pltpu.sync_copy(g_idx(g), ib)
                    fire_gather(ib, 0, 0)
                if not last:
                    pltpu.async_copy(g_idx(g + 1), ib_next, sem_i)
                for t in range(8):
                    st = t % 2
                    if t + 1 < 8:
                        if not (first and t == 0):
                            drain_scatter(1 - st)
                        fire_gather(ib, t + 1, 1 - st)
                    elif not last:
                        drain_scatter(0)
                        pltpu.make_async_copy(
                            g_idx(g + 1), ib_next, sem_i).wait()
                        fire_gather(ib_next, 0, 0)
                    drain_gather(st)
                    sb, tb = bufs[st]

                    def ebody(e4, ecarry):
                        lo = pl.ds(0, 16)
                        hi = pl.ds(16, 16)
                        for u in range(4):
                            e = e4 * 4 + u
                            ex = jnp.exp(tb[e, lo] - sb[e, lo])
                            num = ex * (sb[e, hi] + tb[e, hi])
                            sb[e, lo] = ex
                            sb[e, hi] = num
                        return ecarry
                    lax.fori_loop(0, 32, ebody, None)
                    fire_scatter(ib, t, st)
                if last:
                    drain_scatter(0)
                    drain_scatter(1)

            do_group(0, ibufs[0], ibufs[1], True, ng == 1)
            n_pair = (ng - 3) // 2
            if n_pair > 0:
                def pair(m, carry):
                    g1 = 2 * m + 1
                    do_group(g1, ibufs[1], ibufs[0], False, False)
                    do_group(g1 + 1, ibufs[0], ibufs[1], False, False)
                    return carry
                lax.fori_loop(0, n_pair, pair, None)
            for g in range(max(2 * n_pair + 1, 1), ng):
                do_group(g, ibufs[g % 2], ibufs[1 - g % 2], False,
                         g == ng - 1)
            plsc.subcore_barrier()

            for b in range(nb):
                base = s * rt + b * rb
                pltpu.sync_copy(acc.at[pl.ds(base, rb)], accv)

                def dbody(r, carry):
                    den = accv[r, pl.ds(0, 16)]
                    num = accv[r, pl.ds(16, 16)]
                    obuf[r, pl.ds(0, 16)] = num / (den + 1e-16)
                    return carry
                lax.fori_loop(0, rb, dbody, None)
                pltpu.sync_copy(obuf, out_hbm.at[pl.ds(c * n_acc + base, rb)])
        return ek

    o0 = make_ek(0)(make_body(0))(s_flat, t_flat, idx_all)
    o1 = make_ek(1)(make_body(1))(s_flat, t_flat, idx_all)
    return jnp.stack([o0.reshape(2, n_acc, 16), o1.reshape(2, n_acc, 16)],
                     axis=1).reshape(_NCH * n_acc, 16)


def _build_wt(W_lin, W_src, W_dst, W_pos, b_pos, kp):
    d = W_lin.shape[0]
    bm = jnp.concatenate([W_src, W_pos, jnp.zeros((d, 1), _F32)], axis=1)
    vm = jnp.concatenate([W_lin, -W_pos, jnp.zeros((d, 1), _F32)], axis=1)
    am = jnp.concatenate([W_dst, W_pos, b_pos[:, None]], axis=1)
    qm = jnp.concatenate([jnp.zeros_like(W_src), W_pos, b_pos[:, None]],
                         axis=1)
    dz = bm.shape[1]
    s_rows = jnp.stack([bm.reshape(_NCH, 16, dz), vm.reshape(_NCH, 16, dz)],
                       axis=1).reshape(8 * 16, dz)
    t_rows = jnp.stack([am.reshape(_NCH, 16, dz), qm.reshape(_NCH, 16, dz)],
                       axis=1).reshape(8 * 16, dz)
    wcat = jnp.concatenate([s_rows, t_rows], axis=0)
    return jnp.pad(wcat, ((0, 0), (0, kp - dz))).T


def kernel(pos, edge_index, W_lin1, W_src1, W_dst1, W_pos1, b_pos1,
           W_lin2, W_src2, W_dst2, W_pos2, b_pos2):
    n = pos.shape[0]
    e = edge_index.shape[1]
    np_ = -(-n // _BN) * _BN
    if np_ == n:
        np_ += _BN
    ep = -(-e // _EPAD) * _EPAD

    src = edge_index[0]
    dst = edge_index[1]
    pad_e = ep - e
    if pad_e:
        src = jnp.concatenate([src, jnp.full((pad_e,), n, jnp.int32)])
        dst = jnp.concatenate([dst, jnp.zeros((pad_e,), jnp.int32)])
    nrows = ep // 128
    src_r = src.reshape(nrows, 128)
    dst_r = dst.reshape(nrows, 128)
    idx_all = jnp.stack(
        [jnp.stack([src_r + k * np_, dst_r + k * np_, dst_r], axis=1)
         for k in range(_NCH)], axis=0).reshape(_NCH * nrows, 3, 128)

    ones = jnp.ones((n, 1), _F32)

    def layer(x, W_lin, W_src, W_dst, W_pos, b_pos):
        dz = x.shape[1] + 3 + 1
        kp = -(-dz // 8) * 8
        z = jnp.concatenate([x, pos, ones], axis=1)
        z = jnp.pad(z, ((0, np_ - n), (0, kp - dz)))
        wt = _build_wt(W_lin, W_src, W_dst, W_pos, b_pos, kp)
        s4, t4 = _tc_tables(z, wt)
        t4 = t4.at[:, n, 0:16].set(-1e30)
        out4 = _edge_call(np_, np_, ep,
                          s4.reshape(_NCH * np_, 32),
                          t4.reshape(_NCH * np_, 32), idx_all)
        return (out4.reshape(_NCH, np_, 16)[:, :n]
                .transpose(1, 0, 2).reshape(n, 64))

    x1 = layer(pos, W_lin1, W_src1, W_dst1, W_pos1, b_pos1)
    return layer(x1, W_lin2, W_src2, W_dst2, W_pos2, b_pos2)
